# baseline (speedup 1.0000x reference)
import jax
import jax.numpy as jnp
from jax import lax
from jax.experimental import pallas as pl
from jax.experimental.pallas import tpu as pltpu
from jax.experimental.pallas import tpu_sc as plsc

EMB = 128
OUT_W = 2 * EMB
ROWS = 16384
SC_ROWS = ROWS // 2
NUM_CORES = 2
NUM_SUBCORES = 16
NUM_WORKERS = NUM_CORES * NUM_SUBCORES
ROWS_PER_WORKER = SC_ROWS // NUM_WORKERS
CHUNK = 128
CHUNKS_PER_WORKER = ROWS_PER_WORKER // CHUNK
GROUPS_PER_WORKER = ROWS_PER_WORKER // 16
TC_BLK = 256
TC_BLOCKS = (ROWS - SC_ROWS) // TC_BLK
SC_BLOCKS = SC_ROWS // TC_BLK


def _bucket(lv):
    return ((jnp.minimum(lv, 2) - 1)
            + (jnp.minimum(lv, 4) >> 2)
            + (jnp.minimum(lv, 8) >> 3))


def _encode_body(len_hbm, tab_hbm, out_hbm,
                 len_v, tab_v, boff_v, buf0, buf1, wsem0, wsem1):
    wid = lax.axis_index("s") * NUM_CORES + lax.axis_index("c")
    base = pl.multiple_of(wid * ROWS_PER_WORKER, ROWS_PER_WORKER)

    pltpu.sync_copy(tab_hbm, tab_v)
    pltpu.sync_copy(len_hbm.at[pl.ds(base, ROWS_PER_WORKER)], len_v)

    for g in range(GROUPS_PER_WORKER):
        lv = len_v[pl.ds(g * 16, 16)]
        boff_v[pl.ds(g * 16, 16)] = _bucket(lv) * OUT_W

    iota16 = lax.iota(jnp.int32, 16)

    bufs = (buf0, buf1)
    wsems = (wsem0, wsem1)
    pending = [None, None]
    for c in range(CHUNKS_PER_WORKER):
        b = c % 2
        if pending[b] is not None:
            pending[b].wait()
        buf = bufs[b]

        @plsc.parallel_loop(0, CHUNK, unroll=2)
        def _(r):
            src0 = plsc.load_gather(boff_v, [jnp.broadcast_to(c * CHUNK + r, (16,))])
            src0 = src0 + iota16
            row = jnp.broadcast_to(r, (16,))
            for k in range(OUT_W // 16):
                v = plsc.load_gather(tab_v, [src0 + k * 16])
                plsc.store_scatter(buf, [row, iota16 + k * 16], v)

        pending[b] = pltpu.async_copy(
            buf, out_hbm.at[pl.ds(base + c * CHUNK, CHUNK)], wsems[b])
    pending[0].wait()
    pending[1].wait()


def _tc_body(len_ref, tab_ref, src_ref, out_ref):
    del src_ref
    lv = len_ref[...]
    iota4 = lax.broadcasted_iota(jnp.int32, (1, 4), 1)
    onehot = (_bucket(lv) == iota4).astype(jnp.float32)
    out_ref[...] = jnp.dot(onehot, tab_ref[...],
                           preferred_element_type=jnp.float32)


def kernel(chunks_length, start_pos, genre_id, distance_emb, genre_emb):
    del start_pos
    gid = jnp.asarray(genre_id, jnp.int32)
    genre_row = jnp.take(genre_emb, gid[None], axis=0)
    combined = jnp.concatenate(
        [distance_emb, jnp.broadcast_to(genre_row, (4, EMB))], axis=1)

    mesh = plsc.VectorSubcoreMesh(
        core_axis_name="c", subcore_axis_name="s",
        num_cores=NUM_CORES, num_subcores=NUM_SUBCORES)
    sc_run = pl.kernel(
        _encode_body,
        out_type=jax.ShapeDtypeStruct((ROWS, OUT_W), jnp.float32),
        mesh=mesh,
        compiler_params=pltpu.CompilerParams(needs_layout_passes=False),
        scratch_types=[
            pltpu.VMEM((ROWS_PER_WORKER,), jnp.int32),
            pltpu.VMEM((4 * OUT_W,), jnp.float32),
            pltpu.VMEM((ROWS_PER_WORKER,), jnp.int32),
            pltpu.VMEM((CHUNK, OUT_W), jnp.float32),
            pltpu.VMEM((CHUNK, OUT_W), jnp.float32),
            pltpu.SemaphoreType.DMA,
            pltpu.SemaphoreType.DMA,
        ],
    )
    sc_out = sc_run(chunks_length, combined.reshape(-1))

    tc_fill = pl.pallas_call(
        _tc_body,
        grid=(TC_BLOCKS,),
        in_specs=[
            pl.BlockSpec((TC_BLK, 1), lambda j: (SC_BLOCKS + j, 0)),
            pl.BlockSpec((4, OUT_W), lambda j: (0, 0)),
            pl.BlockSpec(memory_space=pl.ANY),
        ],
        out_specs=pl.BlockSpec((TC_BLK, OUT_W), lambda j: (SC_BLOCKS + j, 0)),
        out_shape=jax.ShapeDtypeStruct((ROWS, OUT_W), jnp.float32),
        input_output_aliases={2: 0},
    )
    return tc_fill(chunks_length.reshape(ROWS, 1), combined, sc_out)


# speedup vs baseline: 1.6283x; 1.6283x over previous
import jax
import jax.numpy as jnp
from jax import lax
from jax.experimental import pallas as pl
from jax.experimental.pallas import tpu as pltpu
from jax.experimental.pallas import tpu_sc as plsc

EMB = 128
OUT_W = 2 * EMB
ROWS = 16384
NUM_CORES = 2
NUM_SUBCORES = 16
NUM_WORKERS = NUM_CORES * NUM_SUBCORES
ROWS_PER_WORKER = ROWS // NUM_WORKERS
CHUNK = 128
CHUNKS_PER_WORKER = ROWS_PER_WORKER // CHUNK
GROUPS_PER_WORKER = ROWS_PER_WORKER // 16


def _bucket(lv):
    return ((jnp.minimum(lv, 2) - 1)
            + (jnp.minimum(lv, 4) >> 2)
            + (jnp.minimum(lv, 8) >> 3))


def _encode_body(len_hbm, tab_hbm, out_hbm,
                 len_v, tab_v, boff_v, buf0, buf1, wsem0, wsem1):
    wid = lax.axis_index("s") * NUM_CORES + lax.axis_index("c")
    base = pl.multiple_of(wid * ROWS_PER_WORKER, ROWS_PER_WORKER)

    pltpu.sync_copy(tab_hbm, tab_v)
    pltpu.sync_copy(len_hbm.at[pl.ds(base, ROWS_PER_WORKER)], len_v)

    for g in range(GROUPS_PER_WORKER):
        lv = len_v[pl.ds(g * 16, 16)]
        boff_v[pl.ds(g * 16, 16)] = _bucket(lv) * OUT_W

    iota16 = lax.iota(jnp.int32, 16)

    bufs = (buf0, buf1)
    wsems = (wsem0, wsem1)
    pending = [None, None]
    for c in range(CHUNKS_PER_WORKER):
        b = c % 2
        if pending[b] is not None:
            pending[b].wait()
        buf = bufs[b]

        @plsc.parallel_loop(0, CHUNK, unroll=4)
        def _(r):
            src0 = plsc.load_gather(boff_v, [jnp.broadcast_to(c * CHUNK + r, (16,))])
            src0 = src0 + iota16
            row = jnp.broadcast_to(r, (16,))
            for k in range(OUT_W // 16):
                v = plsc.load_gather(tab_v, [src0 + k * 16])
                plsc.store_scatter(buf, [row, iota16 + k * 16], v)

        pending[b] = pltpu.async_copy(
            buf, out_hbm.at[pl.ds(base + c * CHUNK, CHUNK)], wsems[b])
    pending[0].wait()
    pending[1].wait()


def kernel(chunks_length, start_pos, genre_id, distance_emb, genre_emb):
    del start_pos
    gid = jnp.asarray(genre_id, jnp.int32)
    genre_row = jnp.take(genre_emb, gid[None], axis=0)
    combined = jnp.concatenate(
        [distance_emb, jnp.broadcast_to(genre_row, (4, EMB))], axis=1)

    mesh = plsc.VectorSubcoreMesh(
        core_axis_name="c", subcore_axis_name="s",
        num_cores=NUM_CORES, num_subcores=NUM_SUBCORES)
    run = pl.kernel(
        _encode_body,
        out_type=jax.ShapeDtypeStruct((ROWS, OUT_W), jnp.float32),
        mesh=mesh,
        compiler_params=pltpu.CompilerParams(needs_layout_passes=False),
        scratch_types=[
            pltpu.VMEM((ROWS_PER_WORKER,), jnp.int32),
            pltpu.VMEM((4 * OUT_W,), jnp.float32),
            pltpu.VMEM((ROWS_PER_WORKER,), jnp.int32),
            pltpu.VMEM((CHUNK, OUT_W), jnp.float32),
            pltpu.VMEM((CHUNK, OUT_W), jnp.float32),
            pltpu.SemaphoreType.DMA,
            pltpu.SemaphoreType.DMA,
        ],
    )
    return run(chunks_length, combined.reshape(-1))


# speedup vs baseline: 1.7207x; 1.0567x over previous
import jax
import jax.numpy as jnp
from jax import lax
from jax.experimental import pallas as pl
from jax.experimental.pallas import tpu as pltpu
from jax.experimental.pallas import tpu_sc as plsc

EMB = 128
OUT_W = 2 * EMB
ROWS = 16384
NUM_CORES = 2
NUM_SUBCORES = 16
NUM_WORKERS = NUM_CORES * NUM_SUBCORES
ROWS_PER_WORKER = ROWS // NUM_WORKERS
CHUNK = 128
CHUNKS_PER_WORKER = ROWS_PER_WORKER // CHUNK
GROUPS_PER_WORKER = ROWS_PER_WORKER // 16


def _bucket(lv):
    return ((jnp.minimum(lv, 2) - 1)
            + (jnp.minimum(lv, 4) >> 2)
            + (jnp.minimum(lv, 8) >> 3))


def _encode_body(len_hbm, tab_hbm, out_hbm,
                 len_v, tab_v, boff_v, buf0, buf1, wsem0, wsem1):
    wid = lax.axis_index("s") * NUM_CORES + lax.axis_index("c")
    base = pl.multiple_of(wid * ROWS_PER_WORKER, ROWS_PER_WORKER)

    pltpu.sync_copy(tab_hbm, tab_v)
    pltpu.sync_copy(len_hbm.at[pl.ds(base, ROWS_PER_WORKER)], len_v)

    for g in range(GROUPS_PER_WORKER):
        lv = len_v[pl.ds(g * 16, 16)]
        boff_v[pl.ds(g * 16, 16)] = _bucket(lv) * OUT_W

    iota16 = lax.iota(jnp.int32, 16)

    bufs = (buf0, buf1)
    wsems = (wsem0, wsem1)
    pending = [None, None]
    for c in range(CHUNKS_PER_WORKER):
        b = c % 2
        if pending[b] is not None:
            pending[b].wait()
        buf = bufs[b]

        @plsc.parallel_loop(0, CHUNK, unroll=2)
        def _(r):
            src0 = plsc.load_gather(boff_v, [jnp.broadcast_to(c * CHUNK + r, (16,))])
            src0 = src0 + iota16
            row = jnp.broadcast_to(r, (16,))
            for k in range(OUT_W // 16):
                v = plsc.load_gather(tab_v, [src0 + k * 16])
                plsc.store_scatter(buf, [row, iota16 + k * 16], v)

        pending[b] = pltpu.async_copy(
            buf, out_hbm.at[pl.ds(base + c * CHUNK, CHUNK)], wsems[b])
    pending[0].wait()
    pending[1].wait()


def kernel(chunks_length, start_pos, genre_id, distance_emb, genre_emb):
    del start_pos
    gid = jnp.asarray(genre_id, jnp.int32)
    genre_row = jnp.take(genre_emb, gid[None], axis=0)
    combined = jnp.concatenate(
        [distance_emb, jnp.broadcast_to(genre_row, (4, EMB))], axis=1)

    mesh = plsc.VectorSubcoreMesh(
        core_axis_name="c", subcore_axis_name="s",
        num_cores=NUM_CORES, num_subcores=NUM_SUBCORES)
    run = pl.kernel(
        _encode_body,
        out_type=jax.ShapeDtypeStruct((ROWS, OUT_W), jnp.float32),
        mesh=mesh,
        compiler_params=pltpu.CompilerParams(needs_layout_passes=False),
        scratch_types=[
            pltpu.VMEM((ROWS_PER_WORKER,), jnp.int32),
            pltpu.VMEM((4 * OUT_W,), jnp.float32),
            pltpu.VMEM((ROWS_PER_WORKER,), jnp.int32),
            pltpu.VMEM((CHUNK, OUT_W), jnp.float32),
            pltpu.VMEM((CHUNK, OUT_W), jnp.float32),
            pltpu.SemaphoreType.DMA,
            pltpu.SemaphoreType.DMA,
        ],
    )
    return run(chunks_length, combined.reshape(-1))
